# baseline (speedup 1.0000x reference)
import functools

import jax
import jax.numpy as jnp
from jax import lax
from jax.experimental import pallas as pl
from jax.experimental.pallas import tpu as pltpu
from jax.experimental.pallas import tpu_sc as plsc

B = 16384
EMB = 64
NUM = 16
HID = 256
TOTAL = NUM + 4 * EMB

_CH = 128


def _build_gather():
    info = plsc.get_sparse_core_info()
    nc, ns = info.num_cores, info.num_subcores
    nw = nc * ns
    bw = B // nw
    n_ch = bw // _CH

    mesh = plsc.VectorSubcoreMesh(core_axis_name="c", subcore_axis_name="s")

    @functools.partial(
        pl.kernel,
        mesh=mesh,
        compiler_params=pltpu.CompilerParams(use_tc_tiling_on_sc=False),
        out_type=[jax.ShapeDtypeStruct((B, EMB), jnp.float32)] * 4,
        scratch_types=[
            pltpu.VMEM((n_ch, _CH), jnp.int32),
            pltpu.VMEM((bw, EMB), jnp.float32),
            pltpu.SemaphoreType.DMA,
        ],
    )
    def gather_kernel(t_item, t_cat, t_brand, t_shop,
                      i_item, i_cat, i_brand, i_shop,
                      o_item, o_cat, o_brand, o_shop,
                      idx_v, rows_v, sem):
        wid = lax.axis_index("s") * nc + lax.axis_index("c")
        base = wid * bw
        for tbl, idx_h, out_h in ((t_item, i_item, o_item),
                                  (t_cat, i_cat, o_cat),
                                  (t_brand, i_brand, o_brand),
                                  (t_shop, i_shop, o_shop)):
            pltpu.sync_copy(idx_h.at[pl.ds(wid * n_ch, n_ch)], idx_v)
            copies = []
            for j in range(n_ch):
                copies.append(pltpu.async_copy(
                    tbl.at[idx_v.at[j]],
                    rows_v.at[pl.ds(j * _CH, _CH)],
                    sem))
            for c in copies:
                c.wait()
            pltpu.sync_copy(rows_v, out_h.at[pl.ds(base, bw)])

    return gather_kernel


_gather = _build_gather()


def _mm_body(num_ref, e1_ref, e2_ref, e3_ref, e4_ref,
             wn_ref, w1_ref, w2_ref, w3_ref, w4_ref, b_ref, out_ref):
    acc = jnp.dot(num_ref[...], wn_ref[...], preferred_element_type=jnp.float32)
    acc += jnp.dot(e1_ref[...], w1_ref[...], preferred_element_type=jnp.float32)
    acc += jnp.dot(e2_ref[...], w2_ref[...], preferred_element_type=jnp.float32)
    acc += jnp.dot(e3_ref[...], w3_ref[...], preferred_element_type=jnp.float32)
    acc += jnp.dot(e4_ref[...], w4_ref[...], preferred_element_type=jnp.float32)
    out_ref[...] = acc + b_ref[...]


_BM = 2048


def _dense(numerical, e1, e2, e3, e4, wn, w1, w2, w3, w4, b2):
    grid = (B // _BM,)
    row_spec = lambda w: pl.BlockSpec((_BM, w), lambda i: (i, 0))
    full = lambda s: pl.BlockSpec(s, lambda i: (0, 0))
    return pl.pallas_call(
        _mm_body,
        grid=grid,
        in_specs=[
            row_spec(NUM), row_spec(EMB), row_spec(EMB), row_spec(EMB),
            row_spec(EMB),
            full((NUM, HID)), full((EMB, HID)), full((EMB, HID)),
            full((EMB, HID)), full((EMB, HID)), full((1, HID)),
        ],
        out_specs=pl.BlockSpec((_BM, HID), lambda i: (i, 0)),
        out_shape=jax.ShapeDtypeStruct((B, HID), jnp.float32),
    )(numerical, e1, e2, e3, e4, wn, w1, w2, w3, w4, b2)


def kernel(numerical, idx_item_id, idx_category_id, idx_brand_id,
           idx_shop_id, emb_item_id, emb_category_id, emb_brand_id,
           emb_shop_id, W, b):
    to_idx = lambda i: i.astype(jnp.int32).reshape(-1, _CH)
    i_item = to_idx(idx_item_id)
    i_cat = to_idx(idx_category_id)
    i_brand = to_idx(idx_brand_id)
    i_shop = to_idx(idx_shop_id)
    e1, e2, e3, e4 = _gather(emb_item_id, emb_category_id, emb_brand_id,
                             emb_shop_id, i_item, i_cat, i_brand, i_shop)
    wt = W.T
    wn = wt[:NUM]
    w1 = wt[NUM:NUM + EMB]
    w2 = wt[NUM + EMB:NUM + 2 * EMB]
    w3 = wt[NUM + 2 * EMB:NUM + 3 * EMB]
    w4 = wt[NUM + 3 * EMB:]
    return _dense(numerical, e1, e2, e3, e4, wn, w1, w2, w3, w4,
                  b.reshape(1, HID))
